# Initial kernel scaffold; baseline (speedup 1.0000x reference)
import jax
import jax.numpy as jnp
from jax.experimental import pallas as pl


def kernel(hidden_states, router_w, Wg, Wu, Wd):
    raise NotImplementedError("write your pallas kernel here")


# speedup vs baseline: 2.4299x; 2.4299x over previous
import functools

import jax
import jax.numpy as jnp
from jax import lax
from jax.experimental import pallas as pl
from jax.experimental.pallas import tpu as pltpu

E = 256
H = 1024
I = 64
T = 512
EPG = 8
GRID = E // EPG


def _moe_kernel(x_ref, rw_ref, wg_ref, wu_ref, wd_ref, out_ref,
                wfull_ref, xbf_ref):
    step = pl.program_id(0)

    @pl.when(step == 0)
    def _router():
        x = x_ref[...]
        xbf_ref[...] = x.astype(jnp.bfloat16)
        rw = rw_ref[...]
        logits = lax.dot_general(
            rw, x, (((1,), (1,)), ((), ())),
            preferred_element_type=jnp.float32)
        iota = lax.broadcasted_iota(jnp.int32, (E, T), 0)
        m1 = jnp.max(logits, axis=0, keepdims=True)
        cand1 = jnp.where(logits == m1, iota, E)
        i1 = jnp.min(cand1, axis=0, keepdims=True)
        oh1 = iota == i1
        lmask = jnp.where(oh1, jnp.float32(-1e30), logits)
        m2 = jnp.max(lmask, axis=0, keepdims=True)
        cand2 = jnp.where(lmask == m2, iota, E)
        i2 = jnp.min(cand2, axis=0, keepdims=True)
        oh2 = iota == i2
        lse = m1 + jnp.log(jnp.sum(jnp.exp(logits - m1), axis=0,
                                   keepdims=True))
        p1 = jnp.exp(m1 - lse)
        p2 = jnp.exp(m2 - lse)
        d = jnp.exp(p2 - p1)
        s1 = 1.0 / (1.0 + d)
        s2 = 1.0 - s1
        wfull_ref[...] = (jnp.where(oh1, s1, 0.0)
                          + jnp.where(oh2, s2, 0.0)).astype(jnp.float32)

    xbf = xbf_ref[...]
    wg = wg_ref[...].astype(jnp.bfloat16).reshape(EPG * I, H)
    wu = wu_ref[...].astype(jnp.bfloat16).reshape(EPG * I, H)
    g = lax.dot_general(xbf, wg, (((1,), (1,)), ((), ())),
                        preferred_element_type=jnp.float32)
    u = lax.dot_general(xbf, wu, (((1,), (1,)), ((), ())),
                        preferred_element_type=jnp.float32)
    act = (g * (1.0 / (1.0 + jnp.exp(-g)))) * u

    wsl = wfull_ref[pl.ds(step * EPG, EPG), :]
    ri = lax.broadcasted_iota(jnp.int32, (EPG, EPG * I), 0)
    ci = lax.broadcasted_iota(jnp.int32, (EPG, EPG * I), 1) // I
    expand = (ri == ci).astype(jnp.float32)
    wexp = lax.dot_general(wsl, expand, (((0,), (0,)), ((), ())),
                           preferred_element_type=jnp.float32)
    actw = (act * wexp).astype(jnp.bfloat16)

    y = jnp.zeros((T, H), dtype=jnp.float32)
    for j in range(EPG):
        wdj = wd_ref[j].astype(jnp.bfloat16)
        y = y + lax.dot_general(actw[:, j * I:(j + 1) * I], wdj,
                                (((1,), (1,)), ((), ())),
                                preferred_element_type=jnp.float32)

    @pl.when(step == 0)
    def _init():
        out_ref[...] = y

    @pl.when(step != 0)
    def _acc():
        out_ref[...] += y


@functools.partial(jax.jit, static_argnames=("interpret",))
def _moe(x, router_w, Wg, Wu, Wd, interpret=False):
    return pl.pallas_call(
        _moe_kernel,
        grid=(GRID,),
        in_specs=[
            pl.BlockSpec((T, H), lambda e: (0, 0)),
            pl.BlockSpec((E, H), lambda e: (0, 0)),
            pl.BlockSpec((EPG, I, H), lambda e: (e, 0, 0)),
            pl.BlockSpec((EPG, I, H), lambda e: (e, 0, 0)),
            pl.BlockSpec((EPG, H, I), lambda e: (e, 0, 0)),
        ],
        out_specs=pl.BlockSpec((T, H), lambda e: (0, 0)),
        out_shape=jax.ShapeDtypeStruct((T, H), jnp.float32),
        scratch_shapes=[
            pltpu.VMEM((E, T), jnp.float32),
            pltpu.VMEM((T, H), jnp.bfloat16),
        ],
        interpret=interpret,
    )(x, router_w, Wg, Wu, Wd)


def kernel(hidden_states, router_w, Wg, Wu, Wd):
    b, s, h = hidden_states.shape
    x = hidden_states.reshape(-1, h)
    out = _moe(x, router_w, Wg, Wu, Wd)
    return out.reshape(b, s, h)


# speedup vs baseline: 2.4388x; 1.0037x over previous
import functools

import jax
import jax.numpy as jnp
from jax import lax
from jax.experimental import pallas as pl
from jax.experimental.pallas import tpu as pltpu

E = 256
H = 1024
I = 64
T = 512
EPG = 8
GRID = E // EPG


def _moe_kernel(x_ref, rw_ref, wg_ref, wu_ref, wd_ref, out_ref,
                wfull_ref):
    step = pl.program_id(0)

    @pl.when(step == 0)
    def _router():
        x = x_ref[...]
        rw = rw_ref[...]
        logits = lax.dot_general(
            rw, x, (((1,), (1,)), ((), ())),
            preferred_element_type=jnp.float32)
        iota = lax.broadcasted_iota(jnp.int32, (E, T), 0)
        m1 = jnp.max(logits, axis=0, keepdims=True)
        cand1 = jnp.where(logits == m1, iota, E)
        i1 = jnp.min(cand1, axis=0, keepdims=True)
        oh1 = iota == i1
        lmask = jnp.where(oh1, jnp.float32(-1e30), logits)
        m2 = jnp.max(lmask, axis=0, keepdims=True)
        cand2 = jnp.where(lmask == m2, iota, E)
        i2 = jnp.min(cand2, axis=0, keepdims=True)
        oh2 = iota == i2
        lse = m1 + jnp.log(jnp.sum(jnp.exp(logits - m1), axis=0,
                                   keepdims=True))
        p1 = jnp.exp(m1 - lse)
        p2 = jnp.exp(m2 - lse)
        d = jnp.exp(p2 - p1)
        s1 = 1.0 / (1.0 + d)
        s2 = 1.0 - s1
        wfull_ref[...] = (jnp.where(oh1, s1, 0.0)
                          + jnp.where(oh2, s2, 0.0)).astype(jnp.float32)

    xf = x_ref[...]
    wg = wg_ref[...].reshape(EPG * I, H)
    wu = wu_ref[...].reshape(EPG * I, H)
    g = lax.dot_general(xf, wg, (((1,), (1,)), ((), ())),
                        preferred_element_type=jnp.float32)
    u = lax.dot_general(xf, wu, (((1,), (1,)), ((), ())),
                        preferred_element_type=jnp.float32)
    act = (g * (1.0 / (1.0 + jnp.exp(-g)))) * u

    wsl = wfull_ref[pl.ds(step * EPG, EPG), :]
    ri = lax.broadcasted_iota(jnp.int32, (EPG, EPG * I), 0)
    ci = lax.broadcasted_iota(jnp.int32, (EPG, EPG * I), 1) // I
    expand = (ri == ci).astype(jnp.float32)
    wexp = lax.dot_general(wsl, expand, (((0,), (0,)), ((), ())),
                           preferred_element_type=jnp.float32)
    actw = act * wexp

    y = jnp.zeros((T, H), dtype=jnp.float32)
    for j in range(EPG):
        wdj = wd_ref[j]
        y = y + lax.dot_general(actw[:, j * I:(j + 1) * I], wdj,
                                (((1,), (1,)), ((), ())),
                                preferred_element_type=jnp.float32)

    @pl.when(step == 0)
    def _init():
        out_ref[...] = y

    @pl.when(step != 0)
    def _acc():
        out_ref[...] += y


@functools.partial(jax.jit, static_argnames=("interpret",))
def _moe(x, router_w, Wg, Wu, Wd, interpret=False):
    return pl.pallas_call(
        _moe_kernel,
        grid=(GRID,),
        in_specs=[
            pl.BlockSpec((T, H), lambda e: (0, 0)),
            pl.BlockSpec((E, H), lambda e: (0, 0)),
            pl.BlockSpec((EPG, I, H), lambda e: (e, 0, 0)),
            pl.BlockSpec((EPG, I, H), lambda e: (e, 0, 0)),
            pl.BlockSpec((EPG, H, I), lambda e: (e, 0, 0)),
        ],
        out_specs=pl.BlockSpec((T, H), lambda e: (0, 0)),
        out_shape=jax.ShapeDtypeStruct((T, H), jnp.float32),
        scratch_shapes=[
            pltpu.VMEM((E, T), jnp.float32),
        ],
        interpret=interpret,
    )(x, router_w, Wg, Wu, Wd)


def kernel(hidden_states, router_w, Wg, Wu, Wd):
    b, s, h = hidden_states.shape
    x = hidden_states.reshape(-1, h)
    out = _moe(x, router_w, Wg, Wu, Wd)
    return out.reshape(b, s, h)


# speedup vs baseline: 2.5425x; 1.0425x over previous
import functools

import jax
import jax.numpy as jnp
from jax import lax
from jax.experimental import pallas as pl
from jax.experimental.pallas import tpu as pltpu

E = 256
H = 1024
I = 64
T = 512
EPG = 16
GRID = E // EPG


def _moe_kernel(x_ref, rw_ref, wg_ref, wu_ref, wd_ref, out_ref,
                wfull_ref):
    step = pl.program_id(0)

    @pl.when(step == 0)
    def _router():
        x = x_ref[...]
        rw = rw_ref[...]
        logits = lax.dot_general(
            rw, x, (((1,), (1,)), ((), ())),
            preferred_element_type=jnp.float32)
        iota = lax.broadcasted_iota(jnp.int32, (E, T), 0)
        m1 = jnp.max(logits, axis=0, keepdims=True)
        cand1 = jnp.where(logits == m1, iota, E)
        i1 = jnp.min(cand1, axis=0, keepdims=True)
        oh1 = iota == i1
        lmask = jnp.where(oh1, jnp.float32(-1e30), logits)
        m2 = jnp.max(lmask, axis=0, keepdims=True)
        cand2 = jnp.where(lmask == m2, iota, E)
        i2 = jnp.min(cand2, axis=0, keepdims=True)
        oh2 = iota == i2
        lse = m1 + jnp.log(jnp.sum(jnp.exp(logits - m1), axis=0,
                                   keepdims=True))
        p1 = jnp.exp(m1 - lse)
        p2 = jnp.exp(m2 - lse)
        d = jnp.exp(p2 - p1)
        s1 = 1.0 / (1.0 + d)
        s2 = 1.0 - s1
        wfull_ref[...] = (jnp.where(oh1, s1, 0.0)
                          + jnp.where(oh2, s2, 0.0)).astype(jnp.float32)

    xf = x_ref[...]
    wg = wg_ref[...].reshape(EPG * I, H)
    wu = wu_ref[...].reshape(EPG * I, H)
    g = lax.dot_general(xf, wg, (((1,), (1,)), ((), ())),
                        preferred_element_type=jnp.float32)
    u = lax.dot_general(xf, wu, (((1,), (1,)), ((), ())),
                        preferred_element_type=jnp.float32)
    act = (g * (1.0 / (1.0 + jnp.exp(-g)))) * u

    wsl = wfull_ref[pl.ds(step * EPG, EPG), :]
    ri = lax.broadcasted_iota(jnp.int32, (EPG, EPG * I), 0)
    ci = lax.broadcasted_iota(jnp.int32, (EPG, EPG * I), 1) // I
    expand = (ri == ci).astype(jnp.float32)
    wexp = lax.dot_general(wsl, expand, (((0,), (0,)), ((), ())),
                           preferred_element_type=jnp.float32)
    actw = act * wexp

    y = jnp.zeros((T, H), dtype=jnp.float32)
    for j in range(EPG):
        wdj = wd_ref[j]
        y = y + lax.dot_general(actw[:, j * I:(j + 1) * I], wdj,
                                (((1,), (1,)), ((), ())),
                                preferred_element_type=jnp.float32)

    @pl.when(step == 0)
    def _init():
        out_ref[...] = y

    @pl.when(step != 0)
    def _acc():
        out_ref[...] += y


@functools.partial(jax.jit, static_argnames=("interpret",))
def _moe(x, router_w, Wg, Wu, Wd, interpret=False):
    return pl.pallas_call(
        _moe_kernel,
        grid=(GRID,),
        in_specs=[
            pl.BlockSpec((T, H), lambda e: (0, 0)),
            pl.BlockSpec((E, H), lambda e: (0, 0)),
            pl.BlockSpec((EPG, I, H), lambda e: (e, 0, 0)),
            pl.BlockSpec((EPG, I, H), lambda e: (e, 0, 0)),
            pl.BlockSpec((EPG, H, I), lambda e: (e, 0, 0)),
        ],
        out_specs=pl.BlockSpec((T, H), lambda e: (0, 0)),
        out_shape=jax.ShapeDtypeStruct((T, H), jnp.float32),
        scratch_shapes=[
            pltpu.VMEM((E, T), jnp.float32),
        ],
        interpret=interpret,
    )(x, router_w, Wg, Wu, Wd)


def kernel(hidden_states, router_w, Wg, Wu, Wd):
    b, s, h = hidden_states.shape
    x = hidden_states.reshape(-1, h)
    out = _moe(x, router_w, Wg, Wu, Wd)
    return out.reshape(b, s, h)


# speedup vs baseline: 2.7702x; 1.0896x over previous
import functools

import jax
import jax.numpy as jnp
from jax import lax
from jax.experimental import pallas as pl
from jax.experimental.pallas import tpu as pltpu

E = 256
H = 1024
I = 64
T = 512
EPG = 16
SUB = EPG // 2
GRID = E // EPG
EHALF = E // 2


def _router(x, rw, wfull_ref):
    logits = lax.dot_general(rw, x, (((1,), (1,)), ((), ())),
                             preferred_element_type=jnp.float32)
    iota = lax.broadcasted_iota(jnp.int32, (E, T), 0)
    m1 = jnp.max(logits, axis=0, keepdims=True)
    cand1 = jnp.where(logits == m1, iota, E)
    i1 = jnp.min(cand1, axis=0, keepdims=True)
    oh1 = iota == i1
    lmask = jnp.where(oh1, jnp.float32(-1e30), logits)
    m2 = jnp.max(lmask, axis=0, keepdims=True)
    cand2 = jnp.where(lmask == m2, iota, E)
    i2 = jnp.min(cand2, axis=0, keepdims=True)
    oh2 = iota == i2
    lse = m1 + jnp.log(jnp.sum(jnp.exp(logits - m1), axis=0, keepdims=True))
    p1 = jnp.exp(m1 - lse)
    p2 = jnp.exp(m2 - lse)
    d = jnp.exp(p2 - p1)
    s1 = 1.0 / (1.0 + d)
    s2 = 1.0 - s1
    wfull_ref[...] = jnp.where(oh1, s1, 0.0) + jnp.where(oh2, s2, 0.0)


def _half(xf, wfull_ref, wg_ref, wu_ref, wd_ref, col0):
    wg = wg_ref[0].reshape(SUB * I, H)
    wu = wu_ref[0].reshape(SUB * I, H)
    g = lax.dot_general(xf, wg, (((1,), (1,)), ((), ())),
                        preferred_element_type=jnp.float32)
    u = lax.dot_general(xf, wu, (((1,), (1,)), ((), ())),
                        preferred_element_type=jnp.float32)
    act = (g * (1.0 / (1.0 + jnp.exp(-g)))) * u

    wsl = wfull_ref[pl.ds(col0, SUB), :]
    ri = lax.broadcasted_iota(jnp.int32, (SUB, SUB * I), 0)
    ci = lax.broadcasted_iota(jnp.int32, (SUB, SUB * I), 1) // I
    expand = (ri == ci).astype(jnp.float32)
    wexp = lax.dot_general(wsl, expand, (((0,), (0,)), ((), ())),
                           preferred_element_type=jnp.float32)
    actw = act * wexp

    y = None
    for j in range(SUB):
        wdj = wd_ref[0, j]
        yj = lax.dot_general(actw[:, j * I:(j + 1) * I], wdj,
                             (((1,), (1,)), ((), ())),
                             preferred_element_type=jnp.float32)
        y = yj if y is None else y + yj
    return y


def _moe_kernel(x_ref, rw_ref, wg0_ref, wg1_ref, wu0_ref, wu1_ref,
                wd0_ref, wd1_ref, out_ref, wfull_ref):
    step = pl.program_id(0)

    @pl.when(step == 0)
    def _r():
        _router(x_ref[...], rw_ref[...], wfull_ref)

    xf = x_ref[...]
    ya = _half(xf, wfull_ref, wg0_ref, wu0_ref, wd0_ref, step * SUB)
    yb = _half(xf, wfull_ref, wg1_ref, wu1_ref, wd1_ref,
               EHALF + step * SUB)
    y = ya + yb

    @pl.when(step == 0)
    def _init():
        out_ref[...] = y

    @pl.when(step != 0)
    def _acc():
        out_ref[...] += y


@functools.partial(jax.jit, static_argnames=("interpret",))
def _moe(x, router_w, Wg, Wu, Wd, interpret=False):
    Wg2 = Wg.reshape(2, EHALF, I, H)
    Wu2 = Wu.reshape(2, EHALF, I, H)
    Wd2 = Wd.reshape(2, EHALF, H, I)
    ga = pl.BlockSpec((1, SUB, I, H), lambda e: (0, e, 0, 0))
    gb = pl.BlockSpec((1, SUB, I, H), lambda e: (1, e, 0, 0))
    da = pl.BlockSpec((1, SUB, H, I), lambda e: (0, e, 0, 0))
    db = pl.BlockSpec((1, SUB, H, I), lambda e: (1, e, 0, 0))
    return pl.pallas_call(
        _moe_kernel,
        grid=(EHALF // SUB,),
        in_specs=[
            pl.BlockSpec((T, H), lambda e: (0, 0)),
            pl.BlockSpec((E, H), lambda e: (0, 0)),
            ga, gb, ga, gb, da, db,
        ],
        out_specs=pl.BlockSpec((T, H), lambda e: (0, 0)),
        out_shape=jax.ShapeDtypeStruct((T, H), jnp.float32),
        scratch_shapes=[
            pltpu.VMEM((E, T), jnp.float32),
        ],
        interpret=interpret,
    )(x, router_w, Wg2, Wg2, Wu2, Wu2, Wd2, Wd2)


def kernel(hidden_states, router_w, Wg, Wu, Wd):
    b, s, h = hidden_states.shape
    x = hidden_states.reshape(-1, h)
    out = _moe(x, router_w, Wg, Wu, Wd)
    return out.reshape(b, s, h)


# speedup vs baseline: 3.4133x; 1.2321x over previous
import functools

import jax
import jax.numpy as jnp
from jax import lax
from jax.experimental import pallas as pl
from jax.experimental.pallas import tpu as pltpu

E = 256
H = 1024
I = 64
T = 512
EPG = 16
SUB = EPG // 2
GRID = E // EPG
EHALF = E // 2


def _router(x, rw, wfull_ref):
    logits = lax.dot_general(rw, x, (((1,), (1,)), ((), ())),
                             preferred_element_type=jnp.float32)
    iota = lax.broadcasted_iota(jnp.int32, (E, T), 0)
    m1 = jnp.max(logits, axis=0, keepdims=True)
    cand1 = jnp.where(logits == m1, iota, E)
    i1 = jnp.min(cand1, axis=0, keepdims=True)
    oh1 = iota == i1
    lmask = jnp.where(oh1, jnp.float32(-1e30), logits)
    m2 = jnp.max(lmask, axis=0, keepdims=True)
    cand2 = jnp.where(lmask == m2, iota, E)
    i2 = jnp.min(cand2, axis=0, keepdims=True)
    oh2 = iota == i2
    lse = m1 + jnp.log(jnp.sum(jnp.exp(logits - m1), axis=0, keepdims=True))
    p1 = jnp.exp(m1 - lse)
    p2 = jnp.exp(m2 - lse)
    d = jnp.exp(p2 - p1)
    s1 = 1.0 / (1.0 + d)
    s2 = 1.0 - s1
    wfull_ref[...] = jnp.where(oh1, s1, 0.0) + jnp.where(oh2, s2, 0.0)


def _gate_up(xbf, wfull_ref, wg_ref, wu_ref, col0):
    wg = wg_ref[0].reshape(SUB * I, H).astype(jnp.bfloat16)
    wu = wu_ref[0].reshape(SUB * I, H).astype(jnp.bfloat16)
    g = lax.dot_general(xbf, wg, (((1,), (1,)), ((), ())),
                        preferred_element_type=jnp.float32)
    u = lax.dot_general(xbf, wu, (((1,), (1,)), ((), ())),
                        preferred_element_type=jnp.float32)
    act = (g * (1.0 / (1.0 + jnp.exp(-g)))) * u

    wsl = wfull_ref[pl.ds(col0, SUB), :]
    ri = lax.broadcasted_iota(jnp.int32, (SUB, SUB * I), 0)
    ci = lax.broadcasted_iota(jnp.int32, (SUB, SUB * I), 1) // I
    expand = (ri == ci).astype(jnp.float32)
    wexp = lax.dot_general(wsl, expand, (((0,), (0,)), ((), ())),
                           preferred_element_type=jnp.float32)
    return act * wexp


def _moe_kernel(x_ref, rw_ref, wg0_ref, wg1_ref, wu0_ref, wu1_ref,
                wd0_ref, wd1_ref, out_ref, wfull_ref, xbf_ref, c_ref):
    step = pl.program_id(0)

    @pl.when(step == 0)
    def _r():
        _router(x_ref[...], rw_ref[...], wfull_ref)
        xbf_ref[...] = x_ref[...].astype(jnp.bfloat16)

    xbf = xbf_ref[...]
    aw0 = _gate_up(xbf, wfull_ref, wg0_ref, wu0_ref, step * SUB)
    aw1 = _gate_up(xbf, wfull_ref, wg1_ref, wu1_ref, EHALF + step * SUB)
    actw = jnp.concatenate([aw0, aw1], axis=1).astype(jnp.bfloat16)

    for j in range(SUB):
        c_ref[:, j * I:(j + 1) * I] = wd0_ref[0, j].astype(jnp.bfloat16)
        c_ref[:, (SUB + j) * I:(SUB + j + 1) * I] = (
            wd1_ref[0, j].astype(jnp.bfloat16))

    y = lax.dot_general(actw, c_ref[...], (((1,), (1,)), ((), ())),
                        preferred_element_type=jnp.float32)

    @pl.when(step == 0)
    def _init():
        out_ref[...] = y

    @pl.when(step != 0)
    def _acc():
        out_ref[...] += y


@functools.partial(jax.jit, static_argnames=("interpret",))
def _moe(x, router_w, Wg, Wu, Wd, interpret=False):
    Wg2 = Wg.reshape(2, EHALF, I, H)
    Wu2 = Wu.reshape(2, EHALF, I, H)
    Wd2 = Wd.reshape(2, EHALF, H, I)
    ga = pl.BlockSpec((1, SUB, I, H), lambda e: (0, e, 0, 0))
    gb = pl.BlockSpec((1, SUB, I, H), lambda e: (1, e, 0, 0))
    da = pl.BlockSpec((1, SUB, H, I), lambda e: (0, e, 0, 0))
    db = pl.BlockSpec((1, SUB, H, I), lambda e: (1, e, 0, 0))
    return pl.pallas_call(
        _moe_kernel,
        grid=(EHALF // SUB,),
        in_specs=[
            pl.BlockSpec((T, H), lambda e: (0, 0)),
            pl.BlockSpec((E, H), lambda e: (0, 0)),
            ga, gb, ga, gb, da, db,
        ],
        out_specs=pl.BlockSpec((T, H), lambda e: (0, 0)),
        out_shape=jax.ShapeDtypeStruct((T, H), jnp.float32),
        scratch_shapes=[
            pltpu.VMEM((E, T), jnp.float32),
            pltpu.VMEM((T, H), jnp.bfloat16),
            pltpu.VMEM((H, EPG * I), jnp.bfloat16),
        ],
        interpret=interpret,
    )(x, router_w, Wg2, Wg2, Wu2, Wu2, Wd2, Wd2)


def kernel(hidden_states, router_w, Wg, Wu, Wd):
    b, s, h = hidden_states.shape
    x = hidden_states.reshape(-1, h)
    out = _moe(x, router_w, Wg, Wu, Wd)
    return out.reshape(b, s, h)
